# baseline (speedup 1.0000x reference)
import functools

import jax
import jax.numpy as jnp
from jax import lax
from jax.experimental import pallas as pl
from jax.experimental.pallas import tpu as pltpu
from jax.experimental.pallas import tpu_sc as plsc

B = 32
N = 1024
D = 768
F = 192
K = 64
C = 28
CP = 32
NH = 8
DH = D // NH

_f32 = jnp.float32
_i32 = jnp.int32


def _scorer_body(x_ref, w1_ref, b1_ref, w2r_ref, b2_ref, out_ref):
    xb = x_ref[0]
    h = jnp.dot(xb, w1_ref[...], preferred_element_type=_f32)
    h = jnp.maximum(h + b1_ref[...], 0.0)
    s = jnp.sum(h * w2r_ref[...], axis=-1) + b2_ref[0, 0]
    out_ref[0, 0] = jax.nn.sigmoid(s)


def _scores(x, w1, b1r, w2r, b2r):
    return pl.pallas_call(
        _scorer_body,
        grid=(B,),
        in_specs=[
            pl.BlockSpec((1, N, D), lambda b: (b, 0, 0)),
            pl.BlockSpec((D, F), lambda b: (0, 0)),
            pl.BlockSpec((1, F), lambda b: (0, 0)),
            pl.BlockSpec((1, F), lambda b: (0, 0)),
            pl.BlockSpec((1, 1), lambda b: (0, 0)),
        ],
        out_specs=pl.BlockSpec((1, 1, N), lambda b: (b, 0, 0)),
        out_shape=jax.ShapeDtypeStruct((B, 1, N), _f32),
    )(x, w1, b1r, w2r, b2r)


def _qpre_body(lab_ref, pw_ref, pb_ref, wq_ref, bq_ref, out_ref):
    lf = jnp.dot(lab_ref[...], pw_ref[...], preferred_element_type=_f32) + pb_ref[...]
    q = jnp.dot(lf, wq_ref[...], preferred_element_type=_f32) + bq_ref[...]
    out_ref[...] = q * (1.0 / (DH ** 0.5))


def _qpre(lab_p, lproj_w, lproj_br, attn_wq, attn_bqr):
    return pl.pallas_call(
        _qpre_body,
        out_shape=jax.ShapeDtypeStruct((CP, D), _f32),
    )(lab_p, lproj_w, lproj_br, attn_wq, attn_bqr)


def _sc_body(scores_hbm, bits_hbm, xflat_hbm, sel_hbm, ssel_hbm,
             sv, iv, idxbuf, idxflat, sselv, rows, sem):
    wid = lax.axis_index("s") * 2 + lax.axis_index("c")

    pltpu.sync_copy(scores_hbm.at[wid], sv)
    pltpu.sync_copy(bits_hbm.at[wid], iv)

    def cnt_gt(t):
        def step(j, acc):
            v = iv[pl.ds(j * 16, 16)]
            return acc + jnp.where(v > t, 1, 0).astype(_i32)
        accv = lax.fori_loop(0, N // 16, step, jnp.zeros((16,), _i32))
        return jnp.sum(accv)

    def bs_step(_, lohi):
        lo, hi = lohi
        mid = lo + (hi - lo) // 2
        c = cnt_gt(mid)
        lo2 = jnp.where(c < K, lo, mid + 1)
        hi2 = jnp.where(c < K, mid, hi)
        return lo2, hi2
    tstar, _ = lax.fori_loop(0, 31, bs_step,
                             (jnp.int32(0), jnp.int32(1 << 30)))

    c1 = cnt_gt(tstar)
    need = K - c1

    def sel_step(j, carry):
        pos, eqseen = carry
        v = iv[pl.ds(j * 16, 16)]
        gt = v > tstar
        eq = v == tstar
        eqrank = eqseen + plsc.cumsum(eq.astype(_i32)) - 1
        m = gt | (eq & (eqrank < need))
        mi = m.astype(_i32)
        outpos = pos + plsc.cumsum(mi) - 1
        idx = lax.iota(_i32, 16) + j * 16
        plsc.store_scatter(idxbuf, [outpos], idx, mask=m)
        return pos + jnp.sum(mi), eqseen + jnp.sum(eq.astype(_i32))
    lax.fori_loop(0, N // 16, sel_step, (jnp.int32(0), jnp.int32(0)))

    def fin_step(t, _):
        iv16 = idxbuf[pl.ds(t * 16, 16)]
        sselv[pl.ds(t * 16, 16)] = plsc.load_gather(sv, [iv16])
        idxflat[pl.ds(t * 16, 16)] = iv16 + wid * N
        return 0
    lax.fori_loop(0, K // 16, fin_step, 0)

    pltpu.async_copy(xflat_hbm.at[idxflat], rows, sem).wait()

    pltpu.sync_copy(rows, sel_hbm.at[wid])
    pltpu.sync_copy(sselv, ssel_hbm.at[wid])


def _topk_gather_sc(scores2d, bits2d, xflat):
    mesh = plsc.VectorSubcoreMesh(core_axis_name="c", subcore_axis_name="s")
    kern = functools.partial(
        pl.kernel,
        mesh=mesh,
        out_type=[
            jax.ShapeDtypeStruct((B, K, D), _f32),
            jax.ShapeDtypeStruct((B, K), _f32),
        ],
        scratch_types=[
            pltpu.VMEM((N,), _f32),
            pltpu.VMEM((N,), _i32),
            pltpu.VMEM((K,), _i32),
            pltpu.VMEM((K,), _i32),
            pltpu.VMEM((K,), _f32),
            pltpu.VMEM((K, D), _f32),
            pltpu.SemaphoreType.DMA,
        ],
        compiler_params=pltpu.CompilerParams(needs_layout_passes=False),
    )(_sc_body)
    return kern(scores2d, bits2d, xflat)


def _layernorm(x, g, b):
    m = jnp.mean(x, axis=-1, keepdims=True)
    v = jnp.mean((x - m) ** 2, axis=-1, keepdims=True)
    return (x - m) / jnp.sqrt(v + 1e-5) * g + b


def _refine_body(sel_ref, ssel_ref, qpre_ref,
                 g0w_ref, g0b_ref, g0g_ref, g0e_ref,
                 g1w_ref, g1b_ref, g1g_ref, g1e_ref,
                 wk_ref, bk_ref, wv_ref, bv_ref, wo_ref, bo_ref,
                 lng_ref, lnb_ref, hw_ref, hb_ref, out_ref):
    sc = ssel_ref[0, 0]
    g = sel_ref[0] * sc[:, None]

    nrm = jnp.maximum(jnp.sqrt(jnp.sum(g * g, axis=-1, keepdims=True)), 1e-12)
    fn = g / nrm
    adj = lax.dot_general(fn, fn, (((1,), (1,)), ((), ())),
                          preferred_element_type=_f32)
    adj = jax.nn.softmax(adj * 10.0, axis=-1)

    for (w_ref, b_ref, ga_ref, be_ref) in ((g0w_ref, g0b_ref, g0g_ref, g0e_ref),
                                           (g1w_ref, g1b_ref, g1g_ref, g1e_ref)):
        o = jnp.dot(adj, g, preferred_element_type=_f32)
        o = jnp.dot(o, w_ref[...], preferred_element_type=_f32) + b_ref[...]
        o = jnp.maximum(_layernorm(o, ga_ref[...], be_ref[...]), 0.0)
        g = g + o

    km = jnp.dot(g, wk_ref[...], preferred_element_type=_f32) + bk_ref[...]
    vm = jnp.dot(g, wv_ref[...], preferred_element_type=_f32) + bv_ref[...]
    q = qpre_ref[...]

    lane = lax.broadcasted_iota(_i32, (1, D), 1)
    head_of_lane = lane // DH
    ao = jnp.zeros((CP, D), _f32)
    for h in range(NH):
        mh = (head_of_lane == h).astype(_f32)
        att = lax.dot_general(q * mh, km * mh, (((1,), (1,)), ((), ())),
                              preferred_element_type=_f32)
        att = jax.nn.softmax(att, axis=-1)
        ao = ao + jnp.dot(att, vm * mh, preferred_element_type=_f32)

    ao = jnp.dot(ao, wo_ref[...], preferred_element_type=_f32) + bo_ref[...]
    ao = _layernorm(ao, lng_ref[...], lnb_ref[...])
    logits = jnp.sum(ao * hw_ref[...], axis=-1) + hb_ref[0, 0]
    out_ref[0, 0] = logits


def _refine(sel, ssel3, qpre, g0w, g0b, g0g, g0e, g1w, g1b, g1g, g1e,
            wk, bk, wv, bv, wo, bo, lng, lnb, hw, hb):
    row = lambda b: (0, 0)
    return pl.pallas_call(
        _refine_body,
        grid=(B,),
        in_specs=[
            pl.BlockSpec((1, K, D), lambda b: (b, 0, 0)),
            pl.BlockSpec((1, 1, K), lambda b: (b, 0, 0)),
            pl.BlockSpec((CP, D), row),
            pl.BlockSpec((D, D), row), pl.BlockSpec((1, D), row),
            pl.BlockSpec((1, D), row), pl.BlockSpec((1, D), row),
            pl.BlockSpec((D, D), row), pl.BlockSpec((1, D), row),
            pl.BlockSpec((1, D), row), pl.BlockSpec((1, D), row),
            pl.BlockSpec((D, D), row), pl.BlockSpec((1, D), row),
            pl.BlockSpec((D, D), row), pl.BlockSpec((1, D), row),
            pl.BlockSpec((D, D), row), pl.BlockSpec((1, D), row),
            pl.BlockSpec((1, D), row), pl.BlockSpec((1, D), row),
            pl.BlockSpec((1, D), row), pl.BlockSpec((1, 1), row),
        ],
        out_specs=pl.BlockSpec((1, 1, CP), lambda b: (b, 0, 0)),
        out_shape=jax.ShapeDtypeStruct((B, 1, CP), _f32),
    )(sel, ssel3, qpre, g0w, g0b, g0g, g0e, g1w, g1b, g1g, g1e,
      wk, bk, wv, bv, wo, bo, lng, lnb, hw, hb)


def kernel(x, label_embeddings, scorer_w1, scorer_b1, scorer_w2, scorer_b2,
           gcn0_w, gcn0_b, gcn0_g, gcn0_be, gcn1_w, gcn1_b, gcn1_g, gcn1_be,
           lproj_w, lproj_b, attn_wq, attn_wk, attn_wv, attn_bq, attn_bk,
           attn_bv, attn_wo, attn_bo, ln_g, ln_b, head_w, head_b):
    r = lambda a: a.reshape(1, -1)

    scores = _scores(x, scorer_w1, r(scorer_b1), scorer_w2.reshape(1, F),
                     scorer_b2.reshape(1, 1))

    lab_p = jnp.zeros((CP, label_embeddings.shape[1]), _f32).at[:C].set(
        label_embeddings)
    qpre = _qpre(lab_p, lproj_w, r(lproj_b), attn_wq, r(attn_bq))

    scores2d = scores.reshape(B, N)
    bits2d = lax.bitcast_convert_type(scores2d, _i32)
    sel, ssel = _topk_gather_sc(scores2d, bits2d, x.reshape(B * N, D))

    out = _refine(sel, ssel.reshape(B, 1, K), qpre,
                  gcn0_w, r(gcn0_b), r(gcn0_g), r(gcn0_be),
                  gcn1_w, r(gcn1_b), r(gcn1_g), r(gcn1_be),
                  attn_wk, r(attn_bk), attn_wv, r(attn_bv),
                  attn_wo, r(attn_bo), r(ln_g), r(ln_b),
                  head_w.reshape(1, D), head_b.reshape(1, 1))
    return out[:, 0, :C]


# speedup vs baseline: 1.4005x; 1.4005x over previous
import functools

import jax
import jax.numpy as jnp
from jax import lax
from jax.experimental import pallas as pl
from jax.experimental.pallas import tpu as pltpu
from jax.experimental.pallas import tpu_sc as plsc

B = 32
N = 1024
D = 768
F = 192
K = 64
C = 28
CP = 32
NH = 8
DH = D // NH

_f32 = jnp.float32
_i32 = jnp.int32


_SB = 2


def _scorer_body(x_ref, w1_ref, b1_ref, w2r_ref, b2_ref, out_ref):
    xb = x_ref[...].reshape(_SB * N, D).astype(jnp.bfloat16)
    w1b = w1_ref[...].astype(jnp.bfloat16)
    h = jnp.dot(xb, w1b, preferred_element_type=_f32)
    h = jnp.maximum(h + b1_ref[...], 0.0)
    hb = h.astype(jnp.bfloat16).astype(_f32)
    w2b = w2r_ref[...].astype(jnp.bfloat16).astype(_f32)
    s = jnp.sum(hb * w2b, axis=-1) + b2_ref[0, 0]
    out_ref[...] = jax.nn.sigmoid(s).reshape(_SB, 1, N)


def _scores(x, w1, b1r, w2r, b2r):
    return pl.pallas_call(
        _scorer_body,
        grid=(B // _SB,),
        in_specs=[
            pl.BlockSpec((_SB, N, D), lambda b: (b, 0, 0)),
            pl.BlockSpec((D, F), lambda b: (0, 0)),
            pl.BlockSpec((1, F), lambda b: (0, 0)),
            pl.BlockSpec((1, F), lambda b: (0, 0)),
            pl.BlockSpec((1, 1), lambda b: (0, 0)),
        ],
        out_specs=pl.BlockSpec((_SB, 1, N), lambda b: (b, 0, 0)),
        out_shape=jax.ShapeDtypeStruct((B, 1, N), _f32),
    )(x, w1, b1r, w2r, b2r)


def _b16(a):
    return a.astype(jnp.bfloat16)


def _qpre_body(lab_ref, pw_ref, pb_ref, wq_ref, bq_ref, out_ref):
    lf = jnp.dot(_b16(lab_ref[...]), _b16(pw_ref[...]),
                 preferred_element_type=_f32) + pb_ref[...]
    q = jnp.dot(_b16(lf), _b16(wq_ref[...]),
                preferred_element_type=_f32) + bq_ref[...]
    out_ref[...] = q


def _qpre(lab_p, lproj_w, lproj_br, attn_wq, attn_bqr):
    return pl.pallas_call(
        _qpre_body,
        out_shape=jax.ShapeDtypeStruct((CP, D), _f32),
    )(lab_p, lproj_w, lproj_br, attn_wq, attn_bqr)


def _sc_body(scores_hbm, bits_hbm, xflat_hbm, sel_hbm, ssel_hbm,
             sv, iv, idxbuf, idxflat, sselv, rows, sem):
    wid = lax.axis_index("s") * 2 + lax.axis_index("c")

    pltpu.sync_copy(scores_hbm.at[wid], sv)
    pltpu.sync_copy(bits_hbm.at[wid], iv)

    def cnt_gt(t):
        def step(j, acc):
            v = iv[pl.ds(j * 16, 16)]
            return acc + jnp.where(v > t, 1, 0).astype(_i32)
        accv = lax.fori_loop(0, N // 16, step, jnp.zeros((16,), _i32))
        return jnp.sum(accv)

    def bs_step(_, lohi):
        lo, hi = lohi
        mid = lo + (hi - lo) // 2
        c = cnt_gt(mid)
        lo2 = jnp.where(c < K, lo, mid + 1)
        hi2 = jnp.where(c < K, mid, hi)
        return lo2, hi2
    tstar, _ = lax.fori_loop(0, 31, bs_step,
                             (jnp.int32(0), jnp.int32(1 << 30)))

    c1 = cnt_gt(tstar)
    need = K - c1

    def sel_step(j, carry):
        pos, eqseen = carry
        v = iv[pl.ds(j * 16, 16)]
        gt = v > tstar
        eq = v == tstar
        eqrank = eqseen + plsc.cumsum(eq.astype(_i32)) - 1
        m = gt | (eq & (eqrank < need))
        mi = m.astype(_i32)
        outpos = pos + plsc.cumsum(mi) - 1
        idx = lax.iota(_i32, 16) + j * 16
        plsc.store_scatter(idxbuf, [outpos], idx, mask=m)
        return pos + jnp.sum(mi), eqseen + jnp.sum(eq.astype(_i32))
    lax.fori_loop(0, N // 16, sel_step, (jnp.int32(0), jnp.int32(0)))

    def fin_step(t, _):
        iv16 = idxbuf[pl.ds(t * 16, 16)]
        sselv[pl.ds(t * 16, 16)] = plsc.load_gather(sv, [iv16])
        idxflat[pl.ds(t * 16, 16)] = iv16 + wid * N
        return 0
    lax.fori_loop(0, K // 16, fin_step, 0)

    pltpu.async_copy(xflat_hbm.at[idxflat], rows, sem).wait()

    pltpu.sync_copy(rows, sel_hbm.at[wid])
    pltpu.sync_copy(sselv, ssel_hbm.at[wid])


def _topk_gather_sc(scores2d, bits2d, xflat):
    mesh = plsc.VectorSubcoreMesh(core_axis_name="c", subcore_axis_name="s")
    kern = functools.partial(
        pl.kernel,
        mesh=mesh,
        out_type=[
            jax.ShapeDtypeStruct((B, K, D), _f32),
            jax.ShapeDtypeStruct((B, K), _f32),
        ],
        scratch_types=[
            pltpu.VMEM((N,), _f32),
            pltpu.VMEM((N,), _i32),
            pltpu.VMEM((K,), _i32),
            pltpu.VMEM((K,), _i32),
            pltpu.VMEM((K,), _f32),
            pltpu.VMEM((K, D), _f32),
            pltpu.SemaphoreType.DMA,
        ],
        compiler_params=pltpu.CompilerParams(needs_layout_passes=False),
    )(_sc_body)
    return kern(scores2d, bits2d, xflat)


def _layernorm(x, g, b):
    m = jnp.mean(x, axis=-1, keepdims=True)
    v = jnp.mean((x - m) ** 2, axis=-1, keepdims=True)
    return (x - m) / jnp.sqrt(v + 1e-5) * g + b


_RB = 4


def _refine_body(sel_ref, ssel_ref, qpre_ref,
                 g0w_ref, g0b_ref, g0g_ref, g0e_ref,
                 g1w_ref, g1b_ref, g1g_ref, g1e_ref,
                 wk_ref, bk_ref, wv_ref, bv_ref, wo_ref, bo_ref,
                 lng_ref, lnb_ref, hw_ref, hb_ref, out_ref):
    gs, adjs = [], []
    for i in range(_RB):
        sc = ssel_ref[i, 0]
        g = sel_ref[i] * sc[:, None]
        nrm = jnp.maximum(jnp.sqrt(jnp.sum(g * g, axis=-1, keepdims=True)),
                          1e-12)
        fn = g / nrm
        fnb = _b16(fn)
        adj = lax.dot_general(fnb, fnb, (((1,), (1,)), ((), ())),
                              preferred_element_type=_f32)
        adjs.append(jax.nn.softmax(adj * 10.0, axis=-1))
        gs.append(g)
    gstk = jnp.concatenate(gs, axis=0)

    for (w_ref, b_ref, ga_ref, be_ref) in ((g0w_ref, g0b_ref, g0g_ref, g0e_ref),
                                           (g1w_ref, g1b_ref, g1g_ref, g1e_ref)):
        gb = _b16(gstk)
        o = jnp.concatenate(
            [jnp.dot(_b16(adjs[i]), gb[i * K:(i + 1) * K],
                     preferred_element_type=_f32) for i in range(_RB)], axis=0)
        o = jnp.dot(_b16(o), _b16(w_ref[...]),
                    preferred_element_type=_f32) + b_ref[...]
        o = jnp.maximum(_layernorm(o, ga_ref[...], be_ref[...]), 0.0)
        gstk = gstk + o

    gb = _b16(gstk)
    km = jnp.dot(gb, _b16(wk_ref[...]), preferred_element_type=_f32) + bk_ref[...]
    vm = jnp.dot(gb, _b16(wv_ref[...]), preferred_element_type=_f32) + bv_ref[...]
    q = qpre_ref[...]

    lane = lax.broadcasted_iota(_i32, (1, D), 1)
    head_of_lane = lane // DH
    aos = []
    for i in range(_RB):
        kmi = km[i * K:(i + 1) * K]
        vmi = vm[i * K:(i + 1) * K]
        ao = jnp.zeros((CP, D), _f32)
        for h in range(NH):
            mh = (head_of_lane == h).astype(_f32)
            att = lax.dot_general(_b16(q * mh), _b16(kmi * mh),
                                  (((1,), (1,)), ((), ())),
                                  preferred_element_type=_f32)
            att = jax.nn.softmax(att * (1.0 / (DH ** 0.5)), axis=-1)
            ao = ao + jnp.dot(_b16(att), _b16(vmi * mh),
                              preferred_element_type=_f32)
        aos.append(ao)
    aostk = jnp.concatenate(aos, axis=0)

    aostk = jnp.dot(_b16(aostk), _b16(wo_ref[...]),
                    preferred_element_type=_f32) + bo_ref[...]
    aostk = _layernorm(aostk, lng_ref[...], lnb_ref[...])
    hwb = _b16(hw_ref[...]).astype(_f32)
    logits = jnp.sum(_b16(aostk).astype(_f32) * hwb, axis=-1) + hb_ref[0, 0]
    out_ref[...] = logits.reshape(_RB, 1, CP)


def _refine(sel, ssel3, qpre, g0w, g0b, g0g, g0e, g1w, g1b, g1g, g1e,
            wk, bk, wv, bv, wo, bo, lng, lnb, hw, hb):
    row = lambda b: (0, 0)
    return pl.pallas_call(
        _refine_body,
        grid=(B // _RB,),
        in_specs=[
            pl.BlockSpec((_RB, K, D), lambda b: (b, 0, 0)),
            pl.BlockSpec((_RB, 1, K), lambda b: (b, 0, 0)),
            pl.BlockSpec((CP, D), row),
            pl.BlockSpec((D, D), row), pl.BlockSpec((1, D), row),
            pl.BlockSpec((1, D), row), pl.BlockSpec((1, D), row),
            pl.BlockSpec((D, D), row), pl.BlockSpec((1, D), row),
            pl.BlockSpec((1, D), row), pl.BlockSpec((1, D), row),
            pl.BlockSpec((D, D), row), pl.BlockSpec((1, D), row),
            pl.BlockSpec((D, D), row), pl.BlockSpec((1, D), row),
            pl.BlockSpec((D, D), row), pl.BlockSpec((1, D), row),
            pl.BlockSpec((1, D), row), pl.BlockSpec((1, D), row),
            pl.BlockSpec((1, D), row), pl.BlockSpec((1, 1), row),
        ],
        out_specs=pl.BlockSpec((_RB, 1, CP), lambda b: (b, 0, 0)),
        out_shape=jax.ShapeDtypeStruct((B, 1, CP), _f32),
    )(sel, ssel3, qpre, g0w, g0b, g0g, g0e, g1w, g1b, g1g, g1e,
      wk, bk, wv, bv, wo, bo, lng, lnb, hw, hb)


def kernel(x, label_embeddings, scorer_w1, scorer_b1, scorer_w2, scorer_b2,
           gcn0_w, gcn0_b, gcn0_g, gcn0_be, gcn1_w, gcn1_b, gcn1_g, gcn1_be,
           lproj_w, lproj_b, attn_wq, attn_wk, attn_wv, attn_bq, attn_bk,
           attn_bv, attn_wo, attn_bo, ln_g, ln_b, head_w, head_b):
    r = lambda a: a.reshape(1, -1)

    scores = _scores(x, scorer_w1, r(scorer_b1), scorer_w2.reshape(1, F),
                     scorer_b2.reshape(1, 1))

    lab_p = jnp.zeros((CP, label_embeddings.shape[1]), _f32).at[:C].set(
        label_embeddings)
    qpre = _qpre(lab_p, lproj_w, r(lproj_b), attn_wq, r(attn_bq))

    scores2d = scores.reshape(B, N)
    bits2d = lax.bitcast_convert_type(scores2d, _i32)
    sel, ssel = _topk_gather_sc(scores2d, bits2d, x.reshape(B * N, D))

    out = _refine(sel, ssel.reshape(B, 1, K), qpre,
                  gcn0_w, r(gcn0_b), r(gcn0_g), r(gcn0_be),
                  gcn1_w, r(gcn1_b), r(gcn1_g), r(gcn1_be),
                  attn_wk, r(attn_bk), attn_wv, r(attn_bv),
                  attn_wo, r(attn_bo), r(ln_g), r(ln_b),
                  head_w.reshape(1, D), head_b.reshape(1, 1))
    return out[:, 0, :C]


# speedup vs baseline: 1.4231x; 1.0161x over previous
import functools

import jax
import jax.numpy as jnp
from jax import lax
from jax.experimental import pallas as pl
from jax.experimental.pallas import tpu as pltpu
from jax.experimental.pallas import tpu_sc as plsc

B = 32
N = 1024
D = 768
F = 192
K = 64
C = 28
CP = 32
NH = 8
DH = D // NH

_f32 = jnp.float32
_i32 = jnp.int32


_SB = 4


def _scorer_body(x_ref, w1_ref, b1_ref, w2r_ref, b2_ref, out_ref):
    xb = x_ref[...].reshape(_SB * N, D).astype(jnp.bfloat16)
    w1b = w1_ref[...].astype(jnp.bfloat16)
    h = jnp.dot(xb, w1b, preferred_element_type=_f32)
    h = jnp.maximum(h + b1_ref[...], 0.0)
    hb = h.astype(jnp.bfloat16).astype(_f32)
    w2b = w2r_ref[...].astype(jnp.bfloat16).astype(_f32)
    s = jnp.sum(hb * w2b, axis=-1) + b2_ref[0, 0]
    out_ref[...] = jax.nn.sigmoid(s).reshape(_SB, 1, N)


def _scores(x, w1, b1r, w2r, b2r):
    return pl.pallas_call(
        _scorer_body,
        grid=(B // _SB,),
        in_specs=[
            pl.BlockSpec((_SB, N, D), lambda b: (b, 0, 0)),
            pl.BlockSpec((D, F), lambda b: (0, 0)),
            pl.BlockSpec((1, F), lambda b: (0, 0)),
            pl.BlockSpec((1, F), lambda b: (0, 0)),
            pl.BlockSpec((1, 1), lambda b: (0, 0)),
        ],
        out_specs=pl.BlockSpec((_SB, 1, N), lambda b: (b, 0, 0)),
        out_shape=jax.ShapeDtypeStruct((B, 1, N), _f32),
    )(x, w1, b1r, w2r, b2r)


def _b16(a):
    return a.astype(jnp.bfloat16)


def _qpre_body(lab_ref, pw_ref, pb_ref, wq_ref, bq_ref, out_ref):
    lf = jnp.dot(_b16(lab_ref[...]), _b16(pw_ref[...]),
                 preferred_element_type=_f32) + pb_ref[...]
    q = jnp.dot(_b16(lf), _b16(wq_ref[...]),
                preferred_element_type=_f32) + bq_ref[...]
    out_ref[...] = q


def _qpre(lab_p, lproj_w, lproj_br, attn_wq, attn_bqr):
    return pl.pallas_call(
        _qpre_body,
        out_shape=jax.ShapeDtypeStruct((CP, D), _f32),
    )(lab_p, lproj_w, lproj_br, attn_wq, attn_bqr)


def _sc_body(scores_hbm, bits_hbm, xflat_hbm, sel_hbm, ssel_hbm,
             sv, iv, idxbuf, idxflat, sselv, rows, sem):
    wid = lax.axis_index("s") * 2 + lax.axis_index("c")

    pltpu.sync_copy(scores_hbm.at[wid], sv)
    pltpu.sync_copy(bits_hbm.at[wid], iv)

    def cnt_gt(t):
        def step(j, acc):
            v = iv[pl.ds(j * 16, 16)]
            return acc + jnp.where(v > t, 1, 0).astype(_i32)
        accv = lax.fori_loop(0, N // 16, step, jnp.zeros((16,), _i32))
        return jnp.sum(accv)

    def bs_step(_, lohi):
        lo, hi = lohi
        mid = lo + (hi - lo) // 2
        c = cnt_gt(mid)
        lo2 = jnp.where(c < K, lo, mid + 1)
        hi2 = jnp.where(c < K, mid, hi)
        return lo2, hi2
    tstar, _ = lax.fori_loop(0, 31, bs_step,
                             (jnp.int32(0), jnp.int32(1 << 30)))

    c1 = cnt_gt(tstar)
    need = K - c1

    def sel_step(j, carry):
        pos, eqseen = carry
        v = iv[pl.ds(j * 16, 16)]
        gt = v > tstar
        eq = v == tstar
        eqrank = eqseen + plsc.cumsum(eq.astype(_i32)) - 1
        m = gt | (eq & (eqrank < need))
        mi = m.astype(_i32)
        outpos = pos + plsc.cumsum(mi) - 1
        idx = lax.iota(_i32, 16) + j * 16
        plsc.store_scatter(idxbuf, [outpos], idx, mask=m)
        return pos + jnp.sum(mi), eqseen + jnp.sum(eq.astype(_i32))
    lax.fori_loop(0, N // 16, sel_step, (jnp.int32(0), jnp.int32(0)))

    def fin_step(t, _):
        iv16 = idxbuf[pl.ds(t * 16, 16)]
        sselv[pl.ds(t * 16, 16)] = plsc.load_gather(sv, [iv16])
        idxflat[pl.ds(t * 16, 16)] = iv16 + wid * N
        return 0
    lax.fori_loop(0, K // 16, fin_step, 0)

    pltpu.async_copy(xflat_hbm.at[idxflat], rows, sem).wait()

    pltpu.sync_copy(rows, sel_hbm.at[wid])
    pltpu.sync_copy(sselv, ssel_hbm.at[wid])


def _topk_gather_sc(scores2d, bits2d, xflat):
    mesh = plsc.VectorSubcoreMesh(core_axis_name="c", subcore_axis_name="s")
    kern = functools.partial(
        pl.kernel,
        mesh=mesh,
        out_type=[
            jax.ShapeDtypeStruct((B, K, D), _f32),
            jax.ShapeDtypeStruct((B, K), _f32),
        ],
        scratch_types=[
            pltpu.VMEM((N,), _f32),
            pltpu.VMEM((N,), _i32),
            pltpu.VMEM((K,), _i32),
            pltpu.VMEM((K,), _i32),
            pltpu.VMEM((K,), _f32),
            pltpu.VMEM((K, D), _f32),
            pltpu.SemaphoreType.DMA,
        ],
        compiler_params=pltpu.CompilerParams(needs_layout_passes=False),
    )(_sc_body)
    return kern(scores2d, bits2d, xflat)


def _layernorm(x, g, b):
    m = jnp.mean(x, axis=-1, keepdims=True)
    v = jnp.mean((x - m) ** 2, axis=-1, keepdims=True)
    return (x - m) / jnp.sqrt(v + 1e-5) * g + b


_RB = 8


def _refine_body(sel_ref, ssel_ref, qpre_ref,
                 g0w_ref, g0b_ref, g0g_ref, g0e_ref,
                 g1w_ref, g1b_ref, g1g_ref, g1e_ref,
                 wk_ref, bk_ref, wv_ref, bv_ref, wo_ref, bo_ref,
                 lng_ref, lnb_ref, hw_ref, hb_ref, out_ref):
    gs, adjs = [], []
    for i in range(_RB):
        sc = ssel_ref[i, 0]
        g = sel_ref[i] * sc[:, None]
        nrm = jnp.maximum(jnp.sqrt(jnp.sum(g * g, axis=-1, keepdims=True)),
                          1e-12)
        fn = g / nrm
        fnb = _b16(fn)
        adj = lax.dot_general(fnb, fnb, (((1,), (1,)), ((), ())),
                              preferred_element_type=_f32)
        adjs.append(jax.nn.softmax(adj * 10.0, axis=-1))
        gs.append(g)
    gstk = jnp.concatenate(gs, axis=0)

    for (w_ref, b_ref, ga_ref, be_ref) in ((g0w_ref, g0b_ref, g0g_ref, g0e_ref),
                                           (g1w_ref, g1b_ref, g1g_ref, g1e_ref)):
        gb = _b16(gstk)
        o = jnp.concatenate(
            [jnp.dot(_b16(adjs[i]), gb[i * K:(i + 1) * K],
                     preferred_element_type=_f32) for i in range(_RB)], axis=0)
        o = jnp.dot(_b16(o), _b16(w_ref[...]),
                    preferred_element_type=_f32) + b_ref[...]
        o = jnp.maximum(_layernorm(o, ga_ref[...], be_ref[...]), 0.0)
        gstk = gstk + o

    gb = _b16(gstk)
    km = jnp.dot(gb, _b16(wk_ref[...]), preferred_element_type=_f32) + bk_ref[...]
    vm = jnp.dot(gb, _b16(wv_ref[...]), preferred_element_type=_f32) + bv_ref[...]
    q = qpre_ref[...]

    lane = lax.broadcasted_iota(_i32, (1, D), 1)
    head_of_lane = lane // DH
    aos = []
    for i in range(_RB):
        kmi = km[i * K:(i + 1) * K]
        vmi = vm[i * K:(i + 1) * K]
        ao = jnp.zeros((CP, D), _f32)
        for h in range(NH):
            mh = (head_of_lane == h).astype(_f32)
            att = lax.dot_general(_b16(q * mh), _b16(kmi * mh),
                                  (((1,), (1,)), ((), ())),
                                  preferred_element_type=_f32)
            att = jax.nn.softmax(att * (1.0 / (DH ** 0.5)), axis=-1)
            ao = ao + jnp.dot(_b16(att), _b16(vmi * mh),
                              preferred_element_type=_f32)
        aos.append(ao)
    aostk = jnp.concatenate(aos, axis=0)

    aostk = jnp.dot(_b16(aostk), _b16(wo_ref[...]),
                    preferred_element_type=_f32) + bo_ref[...]
    aostk = _layernorm(aostk, lng_ref[...], lnb_ref[...])
    hwb = _b16(hw_ref[...]).astype(_f32)
    logits = jnp.sum(_b16(aostk).astype(_f32) * hwb, axis=-1) + hb_ref[0, 0]
    out_ref[...] = logits.reshape(_RB, 1, CP)


def _refine(sel, ssel3, qpre, g0w, g0b, g0g, g0e, g1w, g1b, g1g, g1e,
            wk, bk, wv, bv, wo, bo, lng, lnb, hw, hb):
    row = lambda b: (0, 0)
    return pl.pallas_call(
        _refine_body,
        grid=(B // _RB,),
        in_specs=[
            pl.BlockSpec((_RB, K, D), lambda b: (b, 0, 0)),
            pl.BlockSpec((_RB, 1, K), lambda b: (b, 0, 0)),
            pl.BlockSpec((CP, D), row),
            pl.BlockSpec((D, D), row), pl.BlockSpec((1, D), row),
            pl.BlockSpec((1, D), row), pl.BlockSpec((1, D), row),
            pl.BlockSpec((D, D), row), pl.BlockSpec((1, D), row),
            pl.BlockSpec((1, D), row), pl.BlockSpec((1, D), row),
            pl.BlockSpec((D, D), row), pl.BlockSpec((1, D), row),
            pl.BlockSpec((D, D), row), pl.BlockSpec((1, D), row),
            pl.BlockSpec((D, D), row), pl.BlockSpec((1, D), row),
            pl.BlockSpec((1, D), row), pl.BlockSpec((1, D), row),
            pl.BlockSpec((1, D), row), pl.BlockSpec((1, 1), row),
        ],
        out_specs=pl.BlockSpec((_RB, 1, CP), lambda b: (b, 0, 0)),
        out_shape=jax.ShapeDtypeStruct((B, 1, CP), _f32),
    )(sel, ssel3, qpre, g0w, g0b, g0g, g0e, g1w, g1b, g1g, g1e,
      wk, bk, wv, bv, wo, bo, lng, lnb, hw, hb)


def kernel(x, label_embeddings, scorer_w1, scorer_b1, scorer_w2, scorer_b2,
           gcn0_w, gcn0_b, gcn0_g, gcn0_be, gcn1_w, gcn1_b, gcn1_g, gcn1_be,
           lproj_w, lproj_b, attn_wq, attn_wk, attn_wv, attn_bq, attn_bk,
           attn_bv, attn_wo, attn_bo, ln_g, ln_b, head_w, head_b):
    r = lambda a: a.reshape(1, -1)

    scores = _scores(x, scorer_w1, r(scorer_b1), scorer_w2.reshape(1, F),
                     scorer_b2.reshape(1, 1))

    lab_p = jnp.zeros((CP, label_embeddings.shape[1]), _f32).at[:C].set(
        label_embeddings)
    qpre = _qpre(lab_p, lproj_w, r(lproj_b), attn_wq, r(attn_bq))

    scores2d = scores.reshape(B, N)
    bits2d = lax.bitcast_convert_type(scores2d, _i32)
    sel, ssel = _topk_gather_sc(scores2d, bits2d, x.reshape(B * N, D))

    out = _refine(sel, ssel.reshape(B, 1, K), qpre,
                  gcn0_w, r(gcn0_b), r(gcn0_g), r(gcn0_be),
                  gcn1_w, r(gcn1_b), r(gcn1_g), r(gcn1_be),
                  attn_wk, r(attn_bk), attn_wv, r(attn_bv),
                  attn_wo, r(attn_bo), r(ln_g), r(ln_b),
                  head_w.reshape(1, D), head_b.reshape(1, 1))
    return out[:, 0, :C]


# speedup vs baseline: 1.5750x; 1.1068x over previous
import functools

import jax
import jax.numpy as jnp
from jax import lax
from jax.experimental import pallas as pl
from jax.experimental.pallas import tpu as pltpu
from jax.experimental.pallas import tpu_sc as plsc

B = 32
N = 1024
D = 768
F = 192
K = 64
C = 28
CP = 32
NH = 8
DH = D // NH

_f32 = jnp.float32
_i32 = jnp.int32


_SB = 4


def _scorer_body(x_ref, w1_ref, b1_ref, w2r_ref, b2_ref, out_ref):
    xb = x_ref[...].reshape(_SB * N, D).astype(jnp.bfloat16)
    w1b = w1_ref[...].astype(jnp.bfloat16)
    h = jnp.dot(xb, w1b, preferred_element_type=_f32)
    h = jnp.maximum(h + b1_ref[...], 0.0)
    hb = h.astype(jnp.bfloat16).astype(_f32)
    w2b = w2r_ref[...].astype(jnp.bfloat16).astype(_f32)
    s = jnp.sum(hb * w2b, axis=-1) + b2_ref[0, 0]
    out_ref[...] = jax.nn.sigmoid(s).reshape(_SB, 1, N)


def _scores(x, w1, b1r, w2r, b2r):
    return pl.pallas_call(
        _scorer_body,
        grid=(B // _SB,),
        in_specs=[
            pl.BlockSpec((_SB, N, D), lambda b: (b, 0, 0)),
            pl.BlockSpec((D, F), lambda b: (0, 0)),
            pl.BlockSpec((1, F), lambda b: (0, 0)),
            pl.BlockSpec((1, F), lambda b: (0, 0)),
            pl.BlockSpec((1, 1), lambda b: (0, 0)),
        ],
        out_specs=pl.BlockSpec((_SB, 1, N), lambda b: (b, 0, 0)),
        out_shape=jax.ShapeDtypeStruct((B, 1, N), _f32),
    )(x, w1, b1r, w2r, b2r)


def _b16(a):
    return a.astype(jnp.bfloat16)


def _qpre_body(lab_ref, pw_ref, pb_ref, wq_ref, bq_ref, out_ref):
    lf = jnp.dot(_b16(lab_ref[...]), _b16(pw_ref[...]),
                 preferred_element_type=_f32) + pb_ref[...]
    q = jnp.dot(_b16(lf), _b16(wq_ref[...]),
                preferred_element_type=_f32) + bq_ref[...]
    out_ref[...] = q


def _qpre(lab_p, lproj_w, lproj_br, attn_wq, attn_bqr):
    return pl.pallas_call(
        _qpre_body,
        out_shape=jax.ShapeDtypeStruct((CP, D), _f32),
    )(lab_p, lproj_w, lproj_br, attn_wq, attn_bqr)


def _sc_body(scores_hbm, bits_hbm, xflat_hbm, sel_hbm, ssel_hbm,
             sv, iv, idxbuf, idxflat, sselv, rows, sem):
    wid = lax.axis_index("s") * 2 + lax.axis_index("c")

    pltpu.sync_copy(scores_hbm.at[wid], sv)
    pltpu.sync_copy(bits_hbm.at[wid], iv)

    def cnt_gt(t):
        def step(j, acc):
            v = iv[pl.ds(j * 16, 16)]
            return acc + jnp.where(v > t, 1, 0).astype(_i32)
        accv = lax.fori_loop(0, N // 16, step, jnp.zeros((16,), _i32))
        return jnp.sum(accv)

    def bs_step(_, lohi):
        lo, hi = lohi
        mid = lo + (hi - lo) // 2
        c = cnt_gt(mid)
        lo2 = jnp.where(c < K, lo, mid + 1)
        hi2 = jnp.where(c < K, mid, hi)
        return lo2, hi2
    tstar, _ = lax.fori_loop(0, 31, bs_step,
                             (jnp.int32(0), jnp.int32(1 << 30)))

    c1 = cnt_gt(tstar)
    need = K - c1

    def sel_step(j, carry):
        pos, eqseen = carry
        v = iv[pl.ds(j * 16, 16)]
        gt = v > tstar
        eq = v == tstar
        eqrank = eqseen + plsc.cumsum(eq.astype(_i32)) - 1
        m = gt | (eq & (eqrank < need))
        mi = m.astype(_i32)
        outpos = pos + plsc.cumsum(mi) - 1
        idx = lax.iota(_i32, 16) + j * 16
        plsc.store_scatter(idxbuf, [outpos], idx, mask=m)
        return pos + jnp.sum(mi), eqseen + jnp.sum(eq.astype(_i32))
    lax.fori_loop(0, N // 16, sel_step, (jnp.int32(0), jnp.int32(0)))

    def fin_step(t, _):
        iv16 = idxbuf[pl.ds(t * 16, 16)]
        sselv[pl.ds(t * 16, 16)] = plsc.load_gather(sv, [iv16])
        idxflat[pl.ds(t * 16, 16)] = iv16 + wid * N
        return 0
    lax.fori_loop(0, K // 16, fin_step, 0)

    pltpu.async_copy(xflat_hbm.at[idxflat], rows, sem).wait()

    pltpu.sync_copy(rows, sel_hbm.at[wid])
    pltpu.sync_copy(sselv, ssel_hbm.at[wid])


def _topk_gather_sc(scores2d, bits2d, xflat):
    mesh = plsc.VectorSubcoreMesh(core_axis_name="c", subcore_axis_name="s")
    kern = functools.partial(
        pl.kernel,
        mesh=mesh,
        out_type=[
            jax.ShapeDtypeStruct((B, K, D), _f32),
            jax.ShapeDtypeStruct((B, K), _f32),
        ],
        scratch_types=[
            pltpu.VMEM((N,), _f32),
            pltpu.VMEM((N,), _i32),
            pltpu.VMEM((K,), _i32),
            pltpu.VMEM((K,), _i32),
            pltpu.VMEM((K,), _f32),
            pltpu.VMEM((K, D), _f32),
            pltpu.SemaphoreType.DMA,
        ],
        compiler_params=pltpu.CompilerParams(needs_layout_passes=False),
    )(_sc_body)
    return kern(scores2d, bits2d, xflat)


def _layernorm(x, g, b):
    m = jnp.mean(x, axis=-1, keepdims=True)
    v = jnp.mean((x - m) ** 2, axis=-1, keepdims=True)
    return (x - m) / jnp.sqrt(v + 1e-5) * g + b


_RB = 8


def _refine_body(sel_ref, ssel_ref, qpre_ref,
                 g0w_ref, g0b_ref, g0g_ref, g0e_ref,
                 g1w_ref, g1b_ref, g1g_ref, g1e_ref,
                 wk_ref, bk_ref, wv_ref, bv_ref, wo_ref, bo_ref,
                 lng_ref, lnb_ref, hw_ref, hb_ref, out_ref):
    gs, adjs = [], []
    for i in range(_RB):
        sc = ssel_ref[i, 0]
        g = sel_ref[i] * sc[:, None]
        nrm = jnp.maximum(jnp.sqrt(jnp.sum(g * g, axis=-1, keepdims=True)),
                          1e-12)
        fn = g / nrm
        fnb = _b16(fn)
        adj = lax.dot_general(fnb, fnb, (((1,), (1,)), ((), ())),
                              preferred_element_type=_f32)
        adjs.append(jax.nn.softmax(adj * 10.0, axis=-1))
        gs.append(g)
    gstk = jnp.concatenate(gs, axis=0)

    for (w_ref, b_ref, ga_ref, be_ref) in ((g0w_ref, g0b_ref, g0g_ref, g0e_ref),
                                           (g1w_ref, g1b_ref, g1g_ref, g1e_ref)):
        gb = _b16(gstk)
        o = jnp.concatenate(
            [jnp.dot(_b16(adjs[i]), gb[i * K:(i + 1) * K],
                     preferred_element_type=_f32) for i in range(_RB)], axis=0)
        o = jnp.dot(_b16(o), _b16(w_ref[...]),
                    preferred_element_type=_f32) + b_ref[...]
        o = jnp.maximum(_layernorm(o, ga_ref[...], be_ref[...]), 0.0)
        gstk = gstk + o

    gb = _b16(gstk)
    km = jnp.dot(gb, _b16(wk_ref[...]), preferred_element_type=_f32) + bk_ref[...]
    vm = jnp.dot(gb, _b16(wv_ref[...]), preferred_element_type=_f32) + bv_ref[...]
    q = qpre_ref[...]

    lane = lax.broadcasted_iota(_i32, (1, D), 1)
    head_of_lane = lane // DH
    qstkb = _b16(jnp.concatenate(
        [q * (head_of_lane == h).astype(_f32) for h in range(NH)], axis=0))
    rowh = lax.broadcasted_iota(_i32, (NH * CP, 1), 0) // CP
    pomask = (rowh == head_of_lane).astype(_f32)

    isc = 1.0 / (DH ** 0.5)
    aos = []
    for i in range(_RB):
        kmi = _b16(km[i * K:(i + 1) * K])
        vmi = _b16(vm[i * K:(i + 1) * K])
        att = lax.dot_general(kmi, qstkb, (((1,), (1,)), ((), ())),
                              preferred_element_type=_f32)
        att = jax.nn.softmax(att * isc, axis=0)
        po = lax.dot_general(_b16(att), vmi, (((0,), (0,)), ((), ())),
                             preferred_element_type=_f32)
        ao = jnp.sum((po * pomask).reshape(NH, CP, D), axis=0)
        aos.append(ao)
    aostk = jnp.concatenate(aos, axis=0)

    aostk = jnp.dot(_b16(aostk), _b16(wo_ref[...]),
                    preferred_element_type=_f32) + bo_ref[...]
    aostk = _layernorm(aostk, lng_ref[...], lnb_ref[...])
    hwb = _b16(hw_ref[...]).astype(_f32)
    logits = jnp.sum(_b16(aostk).astype(_f32) * hwb, axis=-1) + hb_ref[0, 0]
    out_ref[...] = logits.reshape(_RB, 1, CP)


def _refine(sel, ssel3, qpre, g0w, g0b, g0g, g0e, g1w, g1b, g1g, g1e,
            wk, bk, wv, bv, wo, bo, lng, lnb, hw, hb):
    row = lambda b: (0, 0)
    return pl.pallas_call(
        _refine_body,
        grid=(B // _RB,),
        in_specs=[
            pl.BlockSpec((_RB, K, D), lambda b: (b, 0, 0)),
            pl.BlockSpec((_RB, 1, K), lambda b: (b, 0, 0)),
            pl.BlockSpec((CP, D), row),
            pl.BlockSpec((D, D), row), pl.BlockSpec((1, D), row),
            pl.BlockSpec((1, D), row), pl.BlockSpec((1, D), row),
            pl.BlockSpec((D, D), row), pl.BlockSpec((1, D), row),
            pl.BlockSpec((1, D), row), pl.BlockSpec((1, D), row),
            pl.BlockSpec((D, D), row), pl.BlockSpec((1, D), row),
            pl.BlockSpec((D, D), row), pl.BlockSpec((1, D), row),
            pl.BlockSpec((D, D), row), pl.BlockSpec((1, D), row),
            pl.BlockSpec((1, D), row), pl.BlockSpec((1, D), row),
            pl.BlockSpec((1, D), row), pl.BlockSpec((1, 1), row),
        ],
        out_specs=pl.BlockSpec((_RB, 1, CP), lambda b: (b, 0, 0)),
        out_shape=jax.ShapeDtypeStruct((B, 1, CP), _f32),
    )(sel, ssel3, qpre, g0w, g0b, g0g, g0e, g1w, g1b, g1g, g1e,
      wk, bk, wv, bv, wo, bo, lng, lnb, hw, hb)


def kernel(x, label_embeddings, scorer_w1, scorer_b1, scorer_w2, scorer_b2,
           gcn0_w, gcn0_b, gcn0_g, gcn0_be, gcn1_w, gcn1_b, gcn1_g, gcn1_be,
           lproj_w, lproj_b, attn_wq, attn_wk, attn_wv, attn_bq, attn_bk,
           attn_bv, attn_wo, attn_bo, ln_g, ln_b, head_w, head_b):
    r = lambda a: a.reshape(1, -1)

    scores = _scores(x, scorer_w1, r(scorer_b1), scorer_w2.reshape(1, F),
                     scorer_b2.reshape(1, 1))

    lab_p = jnp.zeros((CP, label_embeddings.shape[1]), _f32).at[:C].set(
        label_embeddings)
    qpre = _qpre(lab_p, lproj_w, r(lproj_b), attn_wq, r(attn_bq))

    scores2d = scores.reshape(B, N)
    bits2d = lax.bitcast_convert_type(scores2d, _i32)
    sel, ssel = _topk_gather_sc(scores2d, bits2d, x.reshape(B * N, D))

    out = _refine(sel, ssel.reshape(B, 1, K), qpre,
                  gcn0_w, r(gcn0_b), r(gcn0_g), r(gcn0_be),
                  gcn1_w, r(gcn1_b), r(gcn1_g), r(gcn1_be),
                  attn_wk, r(attn_bk), attn_wv, r(attn_bv),
                  attn_wo, r(attn_bo), r(ln_g), r(ln_b),
                  head_w.reshape(1, D), head_b.reshape(1, 1))
    return out[:, 0, :C]


# speedup vs baseline: 1.5943x; 1.0123x over previous
import functools

import jax
import jax.numpy as jnp
from jax import lax
from jax.experimental import pallas as pl
from jax.experimental.pallas import tpu as pltpu
from jax.experimental.pallas import tpu_sc as plsc

B = 32
N = 1024
D = 768
F = 192
K = 64
C = 28
CP = 32
NH = 8
DH = D // NH

_f32 = jnp.float32
_i32 = jnp.int32


_SB = 4


def _scorer_body(x_ref, w1_ref, b1_ref, w2r_ref, b2_ref, out_ref):
    xb = x_ref[...].reshape(_SB * N, D).astype(jnp.bfloat16)
    w1b = w1_ref[...].astype(jnp.bfloat16)
    h = jnp.dot(xb, w1b, preferred_element_type=_f32)
    h = jnp.maximum(h + b1_ref[...], 0.0)
    hb = h.astype(jnp.bfloat16).astype(_f32)
    w2b = w2r_ref[...].astype(jnp.bfloat16).astype(_f32)
    s = jnp.sum(hb * w2b, axis=-1) + b2_ref[0, 0]
    out_ref[...] = jax.nn.sigmoid(s).reshape(_SB, 1, N)


def _scores(x, w1, b1r, w2r, b2r):
    return pl.pallas_call(
        _scorer_body,
        grid=(B // _SB,),
        in_specs=[
            pl.BlockSpec((_SB, N, D), lambda b: (b, 0, 0)),
            pl.BlockSpec((D, F), lambda b: (0, 0)),
            pl.BlockSpec((1, F), lambda b: (0, 0)),
            pl.BlockSpec((1, F), lambda b: (0, 0)),
            pl.BlockSpec((1, 1), lambda b: (0, 0)),
        ],
        out_specs=pl.BlockSpec((_SB, 1, N), lambda b: (b, 0, 0)),
        out_shape=jax.ShapeDtypeStruct((B, 1, N), _f32),
    )(x, w1, b1r, w2r, b2r)


def _b16(a):
    return a.astype(jnp.bfloat16)


def _qpre_body(lab_ref, pw_ref, pb_ref, wq_ref, bq_ref, out_ref):
    lf = jnp.dot(_b16(lab_ref[...]), _b16(pw_ref[...]),
                 preferred_element_type=_f32) + pb_ref[...]
    q = jnp.dot(_b16(lf), _b16(wq_ref[...]),
                preferred_element_type=_f32) + bq_ref[...]
    out_ref[...] = jnp.pad(q, ((0, CP - C), (0, 0)))


def _qpre(lab_p, lproj_w, lproj_br, attn_wq, attn_bqr):
    return pl.pallas_call(
        _qpre_body,
        out_shape=jax.ShapeDtypeStruct((CP, D), _f32),
    )(lab_p, lproj_w, lproj_br, attn_wq, attn_bqr)


def _sc_body(scores_hbm, xflat_hbm, sel_hbm, ssel_hbm,
             sv, iv, idxbuf, idxflat, sselv, rows, sem):
    wid = lax.axis_index("s") * 2 + lax.axis_index("c")

    pltpu.sync_copy(scores_hbm.at[wid], sv)

    def bc_step(j, _):
        iv[pl.ds(j * 16, 16)] = lax.bitcast_convert_type(
            sv[pl.ds(j * 16, 16)], _i32)
        return 0
    lax.fori_loop(0, N // 16, bc_step, 0)

    def cnt_gt(t):
        def step(j, acc):
            v = iv[pl.ds(j * 16, 16)]
            return acc + jnp.where(v > t, 1, 0).astype(_i32)
        accv = lax.fori_loop(0, N // 16, step, jnp.zeros((16,), _i32))
        return jnp.sum(accv)

    def bs_step(_, lohi):
        lo, hi = lohi
        mid = lo + (hi - lo) // 2
        c = cnt_gt(mid)
        lo2 = jnp.where(c < K, lo, mid + 1)
        hi2 = jnp.where(c < K, mid, hi)
        return lo2, hi2
    tstar, _ = lax.fori_loop(0, 31, bs_step,
                             (jnp.int32(0), jnp.int32(1 << 30)))

    c1 = cnt_gt(tstar)
    need = K - c1

    def sel_step(j, carry):
        pos, eqseen = carry
        v = iv[pl.ds(j * 16, 16)]
        gt = v > tstar
        eq = v == tstar
        eqrank = eqseen + plsc.cumsum(eq.astype(_i32)) - 1
        m = gt | (eq & (eqrank < need))
        mi = m.astype(_i32)
        outpos = pos + plsc.cumsum(mi) - 1
        idx = lax.iota(_i32, 16) + j * 16
        plsc.store_scatter(idxbuf, [outpos], idx, mask=m)
        return pos + jnp.sum(mi), eqseen + jnp.sum(eq.astype(_i32))
    lax.fori_loop(0, N // 16, sel_step, (jnp.int32(0), jnp.int32(0)))

    def fin_step(t, _):
        iv16 = idxbuf[pl.ds(t * 16, 16)]
        sselv[pl.ds(t * 16, 16)] = plsc.load_gather(sv, [iv16])
        idxflat[pl.ds(t * 16, 16)] = iv16 + wid * N
        return 0
    lax.fori_loop(0, K // 16, fin_step, 0)

    pltpu.async_copy(xflat_hbm.at[idxflat], rows, sem).wait()

    pltpu.sync_copy(rows, sel_hbm.at[wid])
    pltpu.sync_copy(sselv, ssel_hbm.at[wid])


def _topk_gather_sc(scores2d, xflat):
    mesh = plsc.VectorSubcoreMesh(core_axis_name="c", subcore_axis_name="s")
    kern = functools.partial(
        pl.kernel,
        mesh=mesh,
        out_type=[
            jax.ShapeDtypeStruct((B, K, D), _f32),
            jax.ShapeDtypeStruct((B, K), _f32),
        ],
        scratch_types=[
            pltpu.VMEM((N,), _f32),
            pltpu.VMEM((N,), _i32),
            pltpu.VMEM((K,), _i32),
            pltpu.VMEM((K,), _i32),
            pltpu.VMEM((K,), _f32),
            pltpu.VMEM((K, D), _f32),
            pltpu.SemaphoreType.DMA,
        ],
        compiler_params=pltpu.CompilerParams(needs_layout_passes=False),
    )(_sc_body)
    return kern(scores2d, xflat)


def _layernorm(x, g, b):
    m = jnp.mean(x, axis=-1, keepdims=True)
    v = jnp.mean((x - m) ** 2, axis=-1, keepdims=True)
    return (x - m) / jnp.sqrt(v + 1e-5) * g + b


_RB = 8


def _refine_body(sel_ref, ssel_ref, qpre_ref,
                 g0w_ref, g0b_ref, g0g_ref, g0e_ref,
                 g1w_ref, g1b_ref, g1g_ref, g1e_ref,
                 wk_ref, bk_ref, wv_ref, bv_ref, wo_ref, bo_ref,
                 lng_ref, lnb_ref, hw_ref, hb_ref, out_ref):
    gs, adjs = [], []
    for i in range(_RB):
        sc = ssel_ref[i, 0]
        g = sel_ref[i] * sc[:, None]
        nrm = jnp.maximum(jnp.sqrt(jnp.sum(g * g, axis=-1, keepdims=True)),
                          1e-12)
        fn = g / nrm
        fnb = _b16(fn)
        adj = lax.dot_general(fnb, fnb, (((1,), (1,)), ((), ())),
                              preferred_element_type=_f32)
        adjs.append(jax.nn.softmax(adj * 10.0, axis=-1))
        gs.append(g)
    gstk = jnp.concatenate(gs, axis=0)

    for (w_ref, b_ref, ga_ref, be_ref) in ((g0w_ref, g0b_ref, g0g_ref, g0e_ref),
                                           (g1w_ref, g1b_ref, g1g_ref, g1e_ref)):
        gb = _b16(gstk)
        o = jnp.concatenate(
            [jnp.dot(_b16(adjs[i]), gb[i * K:(i + 1) * K],
                     preferred_element_type=_f32) for i in range(_RB)], axis=0)
        o = jnp.dot(_b16(o), _b16(w_ref[...]),
                    preferred_element_type=_f32) + b_ref[...]
        o = jnp.maximum(_layernorm(o, ga_ref[...], be_ref[...]), 0.0)
        gstk = gstk + o

    gb = _b16(gstk)
    km = jnp.dot(gb, _b16(wk_ref[...]), preferred_element_type=_f32) + bk_ref[...]
    vm = jnp.dot(gb, _b16(wv_ref[...]), preferred_element_type=_f32) + bv_ref[...]
    q = qpre_ref[...]

    lane = lax.broadcasted_iota(_i32, (1, D), 1)
    head_of_lane = lane // DH
    qstkb = _b16(jnp.concatenate(
        [q * (head_of_lane == h).astype(_f32) for h in range(NH)], axis=0))
    rowh = lax.broadcasted_iota(_i32, (NH * CP, 1), 0) // CP
    pomask = (rowh == head_of_lane).astype(_f32)

    isc = 1.0 / (DH ** 0.5)
    aos = []
    for i in range(_RB):
        kmi = _b16(km[i * K:(i + 1) * K])
        vmi = _b16(vm[i * K:(i + 1) * K])
        att = lax.dot_general(kmi, qstkb, (((1,), (1,)), ((), ())),
                              preferred_element_type=_f32)
        att = jax.nn.softmax(att * isc, axis=0)
        po = lax.dot_general(_b16(att), vmi, (((0,), (0,)), ((), ())),
                             preferred_element_type=_f32)
        ao = jnp.sum((po * pomask).reshape(NH, CP, D), axis=0)
        aos.append(ao)
    aostk = jnp.concatenate(aos, axis=0)

    aostk = jnp.dot(_b16(aostk), _b16(wo_ref[...]),
                    preferred_element_type=_f32) + bo_ref[...]
    aostk = _layernorm(aostk, lng_ref[...], lnb_ref[...])
    hwb = _b16(hw_ref[...]).astype(_f32)
    logits = jnp.sum(_b16(aostk).astype(_f32) * hwb, axis=-1) + hb_ref[0, 0]
    out_ref[...] = logits.reshape(_RB, 1, CP)


def _refine(sel, ssel3, qpre, g0w, g0b, g0g, g0e, g1w, g1b, g1g, g1e,
            wk, bk, wv, bv, wo, bo, lng, lnb, hw, hb):
    row = lambda b: (0, 0)
    return pl.pallas_call(
        _refine_body,
        grid=(B // _RB,),
        in_specs=[
            pl.BlockSpec((_RB, K, D), lambda b: (b, 0, 0)),
            pl.BlockSpec((_RB, 1, K), lambda b: (b, 0, 0)),
            pl.BlockSpec((CP, D), row),
            pl.BlockSpec((D, D), row), pl.BlockSpec((1, D), row),
            pl.BlockSpec((1, D), row), pl.BlockSpec((1, D), row),
            pl.BlockSpec((D, D), row), pl.BlockSpec((1, D), row),
            pl.BlockSpec((1, D), row), pl.BlockSpec((1, D), row),
            pl.BlockSpec((D, D), row), pl.BlockSpec((1, D), row),
            pl.BlockSpec((D, D), row), pl.BlockSpec((1, D), row),
            pl.BlockSpec((D, D), row), pl.BlockSpec((1, D), row),
            pl.BlockSpec((1, D), row), pl.BlockSpec((1, D), row),
            pl.BlockSpec((1, D), row), pl.BlockSpec((1, 1), row),
        ],
        out_specs=pl.BlockSpec((_RB, 1, CP), lambda b: (b, 0, 0)),
        out_shape=jax.ShapeDtypeStruct((B, 1, CP), _f32),
    )(sel, ssel3, qpre, g0w, g0b, g0g, g0e, g1w, g1b, g1g, g1e,
      wk, bk, wv, bv, wo, bo, lng, lnb, hw, hb)


def kernel(x, label_embeddings, scorer_w1, scorer_b1, scorer_w2, scorer_b2,
           gcn0_w, gcn0_b, gcn0_g, gcn0_be, gcn1_w, gcn1_b, gcn1_g, gcn1_be,
           lproj_w, lproj_b, attn_wq, attn_wk, attn_wv, attn_bq, attn_bk,
           attn_bv, attn_wo, attn_bo, ln_g, ln_b, head_w, head_b):
    r = lambda a: a.reshape(1, -1)

    scores = _scores(x, scorer_w1, r(scorer_b1), scorer_w2.reshape(1, F),
                     scorer_b2.reshape(1, 1))

    qpre = _qpre(label_embeddings, lproj_w, r(lproj_b), attn_wq, r(attn_bq))

    sel, ssel = _topk_gather_sc(scores.reshape(B, N), x.reshape(B * N, D))

    out = _refine(sel, ssel.reshape(B, 1, K), qpre,
                  gcn0_w, r(gcn0_b), r(gcn0_g), r(gcn0_be),
                  gcn1_w, r(gcn1_b), r(gcn1_g), r(gcn1_be),
                  attn_wk, r(attn_bk), attn_wv, r(attn_bv),
                  attn_wo, r(attn_bo), r(ln_g), r(ln_b),
                  head_w.reshape(1, D), head_b.reshape(1, 1))
    return out[:, 0, :C]
